# baseline (speedup 1.0000x reference)
import functools

import jax
import jax.numpy as jnp
from jax import lax
from jax.experimental import pallas as pl
from jax.experimental.pallas import tpu as pltpu
from jax.experimental.pallas import tpu_sc as plsc

_B = 1024
_L = 50
_LP = 56
_V = 1000
_D = 1000
_DP = 1024

_NC = 2
_NS = 16
_NW = _NC * _NS
_B_PER_W = _B // _NW
_NBUF = 2


def _make_gather():
    mesh = plsc.VectorSubcoreMesh(core_axis_name="c", subcore_axis_name="s")

    @functools.partial(
        pl.kernel,
        mesh=mesh,
        out_type=jax.ShapeDtypeStruct((_B, _LP, _DP), jnp.float32),
        scratch_types=[
            pltpu.VMEM((_B_PER_W * _LP,), jnp.int32),
        ]
        + [pltpu.VMEM((_LP, _DP), jnp.float32) for _ in range(_NBUF)]
        + [pltpu.SemaphoreType.DMA for _ in range(2 * _NBUF)],
    )
    def gather_kernel(idx_hbm, tab_hbm, out_hbm, idx_v, *rest):
        buf = rest[:_NBUF]
        gsem = rest[_NBUF:2 * _NBUF]
        wsem = rest[2 * _NBUF:3 * _NBUF]

        sid = lax.axis_index("s")
        wid = sid * _NC + lax.axis_index("c")
        ibase = wid * _B_PER_W * _LP

        pltpu.sync_copy(idx_hbm.at[pl.ds(ibase, _B_PER_W * _LP)], idx_v)

        def start_gather(c, s):
            idx = idx_v.at[pl.ds(c * _LP, _LP)]
            pltpu.async_copy(tab_hbm.at[idx], buf[s], gsem[s])

        def wait_gather(c, s):
            idx = idx_v.at[pl.ds(c * _LP, _LP)]
            pltpu.make_async_copy(tab_hbm.at[idx], buf[s], gsem[s]).wait()

        def start_write(c, s):
            bg = wid * _B_PER_W + c
            pltpu.async_copy(buf[s], out_hbm.at[bg], wsem[s])

        def wait_write(c, s):
            bg = wid * _B_PER_W + c
            pltpu.make_async_copy(buf[s], out_hbm.at[bg], wsem[s]).wait()

        for s in range(_NBUF):
            start_gather(s, s)

        def body(r, carry):
            cb = r * _NBUF
            for s in range(_NBUF):
                wait_gather(cb + s, s)
                start_write(cb + s, s)
            @pl.when(r + 1 < _B_PER_W // _NBUF)
            def _():
                for s in range(_NBUF):
                    wait_write(cb + s, s)
                    start_gather(cb + _NBUF + s, s)
            return carry

        lax.fori_loop(0, _B_PER_W // _NBUF, body, 0)

        for s in range(_NBUF):
            wait_write(_B_PER_W - _NBUF + s, s)

    return gather_kernel


_gather = _make_gather()


def kernel(x, prob):
    xp = jnp.pad(x.astype(jnp.int32), ((0, 0), (0, _LP - _L)))
    flat_idx = xp.reshape(-1)
    tab = jnp.pad(prob, ((0, 0), (0, _DP - _D)))
    out = _gather(flat_idx, tab)
    return out[:, :_L, :_D]
